# baseline (speedup 1.0000x reference)
import functools

import jax
import jax.numpy as jnp
from jax import lax
from jax.experimental import pallas as pl
from jax.experimental.pallas import tpu as pltpu
from jax.experimental.pallas import tpu_sc as plsc

DENORM = (1440, 24, 31, 53, 7, 12)
OFFS = (0, 1440, 1464, 1495, 1548, 1555)
SPATIAL_OFF = 1567
FEATURE_DIM = 3
NODE = 10000
MD = 64
BATCH = 16
ROWS = BATCH * NODE
LANES = 16
CHUNK = 128
NCHUNKS = ROWS // CHUNK
NC, NS = 2, 16
NW = NC * NS
CPW = (NCHUNKS + NW - 1) // NW
NT = 7
NBUF = 4
NOUTER = CPW // NBUF
PADROWS = NW * CPW * CHUNK


def _body(vals_hbm, table_hbm, out_hbm, vals_v, idx_v, acc_v, *sems):
    sem_g = sems[:NBUF]
    sem_o = sems[NBUF:]
    wid = lax.axis_index("s") * NC + lax.axis_index("c")

    def prep(b, j):
        t = j * NW + wid
        row0 = t * CHUNK
        pltpu.sync_copy(vals_hbm.at[:, pl.ds(row0, CHUNK)], vals_v.at[b])
        for v in range(CHUNK // LANES):
            sl = pl.ds(v * LANES, LANES)
            for i in range(6):
                x = vals_v[b, i, sl]
                idx_v[b, i, sl] = (x * DENORM[i]).astype(jnp.int32) + OFFS[i]
            r = row0 + v * LANES + lax.iota(jnp.int32, LANES)
            idx_v[b, 6, sl] = lax.rem(r, NODE) + SPATIAL_OFF
        t_prev = t - NBUF * NW

        @pl.when((t_prev >= 0) & (t_prev < NCHUNKS))
        def _():
            pltpu.make_async_copy(acc_v.at[b], out_hbm.at[pl.ds(0, CHUNK)],
                                  sem_o[b]).wait()

        def zrow(i, c):
            for c4 in range(MD // LANES):
                acc_v[b, i, pl.ds(c4 * LANES, LANES)] = jnp.zeros(
                    (LANES,), jnp.float32)
            return c

        lax.fori_loop(0, CHUNK, zrow, None)
        for jt in range(NT):
            pltpu.async_copy(table_hbm.at[idx_v.at[b, jt]], acc_v.at[b],
                             sem_g[b], add=True)

    def drain(b, j):
        t = j * NW + wid
        row0 = t * CHUNK
        for _ in range(NT):
            pltpu.make_async_copy(table_hbm.at[idx_v.at[b, 0]], acc_v.at[b],
                                  sem_g[b]).wait()

        @pl.when(t < NCHUNKS)
        def _():
            pltpu.async_copy(acc_v.at[b], out_hbm.at[pl.ds(row0, CHUNK)],
                             sem_o[b])

    for b in range(NBUF):
        prep(b, b)

    def outer(k, c):
        for b in range(NBUF):
            j = k * NBUF + b
            drain(b, j)

            @pl.when(k < NOUTER - 1)
            def _():
                prep(b, j + NBUF)
        return c

    lax.fori_loop(0, NOUTER, outer, None)
    for b in range(NBUF):
        t = ((NOUTER - 1) * NBUF + b) * NW + wid

        @pl.when(t < NCHUNKS)
        def _():
            pltpu.make_async_copy(acc_v.at[b], out_hbm.at[pl.ds(0, CHUNK)],
                                  sem_o[b]).wait()


@jax.jit
def kernel(temporal, spatial_prompt, emb0, emb1, emb2, emb3, emb4, emb5):
    vals = temporal[:, -1, :, FEATURE_DIM:FEATURE_DIM + 6]
    vals_t = vals.reshape(ROWS, 6).T
    vals_t = jnp.pad(vals_t, ((0, 0), (0, PADROWS - ROWS)))
    table = jnp.concatenate(
        [emb0, emb1, emb2, emb3, emb4, emb5, spatial_prompt], axis=0)

    mesh = plsc.VectorSubcoreMesh(core_axis_name="c", subcore_axis_name="s",
                                  num_cores=NC, num_subcores=NS)
    scratch = (
        pltpu.VMEM((NBUF, 6, CHUNK), jnp.float32),
        pltpu.VMEM((NBUF, NT, CHUNK), jnp.int32),
        pltpu.VMEM((NBUF, CHUNK, MD), jnp.float32),
    ) + tuple(pltpu.SemaphoreType.DMA for _ in range(2 * NBUF))
    out = pl.kernel(
        _body,
        out_type=jax.ShapeDtypeStruct((ROWS, MD), jnp.float32),
        mesh=mesh,
        scratch_types=scratch,
        compiler_params=pltpu.CompilerParams(use_tc_tiling_on_sc=False),
    )(vals_t, table)
    return out.reshape(BATCH, NODE, MD)


# speedup vs baseline: 3.1633x; 3.1633x over previous
import functools

import jax
import jax.numpy as jnp
from jax import lax
from jax.experimental import pallas as pl
from jax.experimental.pallas import tpu as pltpu
from jax.experimental.pallas import tpu_sc as plsc

DENORM = (1440, 24, 31, 53, 7, 12)
OFFS = (0, 1440, 1464, 1495, 1548, 1555)
TROWS = 1567
FEATURE_DIM = 3
NODE = 10000
MD = 64
BATCH = 16
ROWS = BATCH * NODE
LANES = 16
CHUNK = 128
NCHUNKS = ROWS // CHUNK
NC, NS = 2, 16
NW = NC * NS
CPW = (NCHUNKS + NW - 1) // NW
NSLOT = 2
NOUTER = CPW // NSLOT
PADROWS = NW * CPW * CHUNK


def _body(vals_hbm, table_hbm, spat_hbm, out_hbm,
          table_v, vals_v, acc_v, *sems):
    sem_i = sems[:NSLOT]
    sem_o = sems[NSLOT:]
    wid = lax.axis_index("s") * NC + lax.axis_index("c")

    pltpu.sync_copy(table_hbm, table_v)

    def fire_vals(s, t):
        row0 = t * CHUNK
        pltpu.async_copy(vals_hbm.at[:, pl.ds(row0, CHUNK)], vals_v.at[s],
                         sem_i[s])

    def fire_spat(s, t):
        row0 = t * CHUNK
        m0 = lax.rem(row0, NODE)
        pltpu.async_copy(spat_hbm.at[pl.ds(m0, CHUNK)], acc_v.at[s], sem_i[s])

    def wait_in(s):
        pltpu.make_async_copy(vals_hbm.at[:, pl.ds(0, CHUNK)], vals_v.at[s],
                              sem_i[s]).wait()
        pltpu.make_async_copy(spat_hbm.at[pl.ds(0, CHUNK)], acc_v.at[s],
                              sem_i[s]).wait()

    def wait_out(s):
        pltpu.make_async_copy(acc_v.at[s], out_hbm.at[pl.ds(0, CHUNK)],
                              sem_o[s]).wait()

    def compute(s, t):
        row0 = t * CHUNK

        @plsc.parallel_loop(0, CHUNK // LANES, 1)
        def _(g):
            sg = pl.ds(g * LANES, LANES)
            ivecs = [(vals_v[s, i, sg] * DENORM[i]).astype(jnp.int32) + OFFS[i]
                     for i in range(6)]
            for rl in range(LANES):
                i = g * LANES + rl
                r = [ivecs[jt][rl] for jt in range(6)]
                for c4 in range(MD // LANES):
                    sl = pl.ds(c4 * LANES, LANES)
                    a = acc_v[s, i, sl]
                    for jt in range(6):
                        a = a + table_v[r[jt], sl]
                    acc_v[s, i, sl] = a

        @pl.when(t < NCHUNKS)
        def _():
            pltpu.async_copy(acc_v.at[s], out_hbm.at[pl.ds(row0, CHUNK)],
                             sem_o[s])

    for s in range(NSLOT):
        t = s * NW + wid
        fire_vals(s, t)
        fire_spat(s, t)

    def outer(k, c):
        for s in range(NSLOT):
            j = k * NSLOT + s
            t = j * NW + wid
            wait_in(s)
            compute(s, t)

            @pl.when(k < NOUTER - 1)
            def _():
                t2 = (j + NSLOT) * NW + wid
                fire_vals(s, t2)

                @pl.when(t < NCHUNKS)
                def _():
                    wait_out(s)

                fire_spat(s, t2)
        return c

    lax.fori_loop(0, NOUTER, outer, None)
    for s in range(NSLOT):
        t = ((NOUTER - 1) * NSLOT + s) * NW + wid

        @pl.when(t < NCHUNKS)
        def _():
            wait_out(s)


@jax.jit
def kernel(temporal, spatial_prompt, emb0, emb1, emb2, emb3, emb4, emb5):
    vals = temporal[:, -1, :, FEATURE_DIM:FEATURE_DIM + 6]
    vals_t = vals.reshape(ROWS, 6).T
    vals_t = jnp.pad(vals_t, ((0, 0), (0, PADROWS - ROWS)))
    table = jnp.concatenate([emb0, emb1, emb2, emb3, emb4, emb5], axis=0)
    spat_pad = jnp.concatenate([spatial_prompt, spatial_prompt[:CHUNK]],
                               axis=0)

    mesh = plsc.VectorSubcoreMesh(core_axis_name="c", subcore_axis_name="s",
                                  num_cores=NC, num_subcores=NS)
    scratch = (
        pltpu.VMEM((TROWS, MD), jnp.float32),
        pltpu.VMEM((NSLOT, 6, CHUNK), jnp.float32),
        pltpu.VMEM((NSLOT, CHUNK, MD), jnp.float32),
    ) + tuple(pltpu.SemaphoreType.DMA for _ in range(2 * NSLOT))
    out = pl.kernel(
        _body,
        out_type=jax.ShapeDtypeStruct((ROWS, MD), jnp.float32),
        mesh=mesh,
        scratch_types=scratch,
        compiler_params=pltpu.CompilerParams(use_tc_tiling_on_sc=False),
    )(vals_t, table, spat_pad)
    return out.reshape(BATCH, NODE, MD)
